# Initial kernel scaffold; baseline (speedup 1.0000x reference)
import jax
import jax.numpy as jnp
from jax.experimental import pallas as pl


def kernel(queries, keys):
    raise NotImplementedError("write your pallas kernel here")


# speedup vs baseline: 2.0428x; 2.0428x over previous
import functools

import jax
import jax.numpy as jnp
from jax.experimental import pallas as pl
from jax.experimental.pallas import tpu as pltpu

TOPK = 5
NEG = -1.0e30
BIGI = 2**30


def _body(q_ref, k_ref, ov_ref, oi_ref, cv_ref, ci_ref, *, kb, k_total, n_blocks):
    j = pl.program_id(0)
    nq = q_ref.shape[0]

    @pl.when(j == 0)
    def _init():
        cv_ref[:] = jnp.full(cv_ref.shape, NEG, jnp.float32)
        ci_ref[:] = jnp.full(ci_ref.shape, BIGI, jnp.int32)

    q = q_ref[:]
    qn = q / (jnp.sqrt(jnp.sum(q * q, axis=1, keepdims=True)) + 1e-8)
    k = k_ref[:]
    kn = k / (jnp.sqrt(jnp.sum(k * k, axis=1, keepdims=True)) + 1e-8)
    s = jax.lax.dot_general(
        qn, kn, (((1,), (1,)), ((), ())), preferred_element_type=jnp.float32
    )
    gidx = j * kb + jax.lax.broadcasted_iota(jnp.int32, (nq, kb), 1)
    s = jnp.where(gidx >= k_total, NEG, s)

    bv, bi = [], []
    for _ in range(TOPK):
        m = jnp.max(s, axis=1, keepdims=True)
        it = jnp.min(jnp.where(s == m, gidx, BIGI), axis=1, keepdims=True)
        s = jnp.where(gidx == it, NEG, s)
        bv.append(m)
        bi.append(it)

    cand_v = jnp.concatenate([cv_ref[:]] + bv, axis=1)
    cand_i = jnp.concatenate([ci_ref[:]] + bi, axis=1)
    nv, ni = [], []
    for _ in range(TOPK):
        m = jnp.max(cand_v, axis=1, keepdims=True)
        it = jnp.min(jnp.where(cand_v == m, cand_i, BIGI), axis=1, keepdims=True)
        cand_v = jnp.where(cand_i == it, NEG, cand_v)
        nv.append(m)
        ni.append(it)
    pad = cv_ref.shape[1] - TOPK
    new_cv = jnp.concatenate(nv + [jnp.full((nq, pad), NEG, jnp.float32)], axis=1)
    new_ci = jnp.concatenate(ni + [jnp.full((nq, pad), BIGI, jnp.int32)], axis=1)
    cv_ref[:] = new_cv
    ci_ref[:] = new_ci

    @pl.when(j == n_blocks - 1)
    def _out():
        ov_ref[:] = new_cv
        oi_ref[:] = new_ci


@jax.jit
def kernel(queries, keys):
    nq, d = queries.shape
    k_total = keys.shape[0]
    kb = 2048
    n_blocks = -(-k_total // kb)
    kp = n_blocks * kb
    if kp != k_total:
        keys = jnp.pad(keys, ((0, kp - k_total), (0, 0)))

    cw = 8
    vals, idx = pl.pallas_call(
        functools.partial(_body, kb=kb, k_total=k_total, n_blocks=n_blocks),
        grid=(n_blocks,),
        in_specs=[
            pl.BlockSpec((nq, d), lambda j: (0, 0)),
            pl.BlockSpec((kb, d), lambda j: (j, 0)),
        ],
        out_specs=[
            pl.BlockSpec((nq, cw), lambda j: (0, 0)),
            pl.BlockSpec((nq, cw), lambda j: (0, 0)),
        ],
        out_shape=[
            jax.ShapeDtypeStruct((nq, cw), jnp.float32),
            jax.ShapeDtypeStruct((nq, cw), jnp.int32),
        ],
        scratch_shapes=[
            pltpu.VMEM((nq, cw), jnp.float32),
            pltpu.VMEM((nq, cw), jnp.int32),
        ],
    )(queries, keys)
    return vals[:, :TOPK], idx[:, :TOPK]
